# baseline (speedup 1.0000x reference)
import functools

import jax
import jax.numpy as jnp
from jax import lax
from jax.experimental import pallas as pl
from jax.experimental.pallas import tpu as pltpu
from jax.experimental.pallas import tpu_sc as plsc

H = 1024
I = 1024
E = 64
T = 2048
IB = 512
NI = I // IB
CHUNK = 64
NC, NS = 2, 16
NW = NC * NS
P = 2560
BPW = P // NW
TRASH = T


def _gelu(v):
    return 0.5 * v * (1.0 + jnp.tanh(jnp.sqrt(2.0 / jnp.pi) * (v + 0.044715 * v ** 3)))


_SC_CACHE = {}


def _sc_kernels():
    if "gather" in _SC_CACHE:
        return _SC_CACHE["gather"], _SC_CACHE["scatter"]
    mesh = plsc.VectorSubcoreMesh(core_axis_name="c", subcore_axis_name="s")
    scratch = [
        pltpu.VMEM((BPW,), jnp.int32),
        pltpu.VMEM((BPW, H), jnp.float32),
        pltpu.SemaphoreType.DMA,
    ]

    @functools.partial(
        pl.kernel, mesh=mesh,
        out_type=jax.ShapeDtypeStruct((P, H), jnp.float32),
        scratch_types=scratch,
    )
    def _sc_gather(table_hbm, idx_hbm, out_hbm, idx_v, rows_v, sem):
        wid = lax.axis_index("s") * NC + lax.axis_index("c")
        base = wid * BPW
        pltpu.sync_copy(idx_hbm.at[pl.ds(base, BPW)], idx_v)
        pltpu.async_copy(table_hbm.at[idx_v], rows_v, sem).wait()
        pltpu.sync_copy(rows_v, out_hbm.at[pl.ds(base, BPW)])

    @functools.partial(
        pl.kernel, mesh=mesh,
        out_type=jax.ShapeDtypeStruct((T + P, H), jnp.float32),
        scratch_types=scratch,
    )
    def _sc_scatter(rows_hbm, idx_hbm, out_hbm, idx_v, rows_v, sem):
        wid = lax.axis_index("s") * NC + lax.axis_index("c")
        base = wid * BPW
        pltpu.sync_copy(idx_hbm.at[pl.ds(base, BPW)], idx_v)
        pltpu.sync_copy(rows_hbm.at[pl.ds(base, BPW)], rows_v)
        pltpu.async_copy(rows_v, out_hbm.at[idx_v], sem).wait()

    _SC_CACHE["gather"] = _sc_gather
    _SC_CACHE["scatter"] = _sc_scatter
    return _sc_gather, _sc_scatter


def _tc_body(poff_ref, xs_ref, ws_ref, wg_ref, wu_ref, wd_ref, ys_ref):
    e = pl.program_id(0)
    it = pl.program_id(1)
    start = poff_ref[e]
    size = poff_ref[e + 1] - start
    nch = (size + CHUNK - 1) // CHUNK
    wg = wg_ref[0]
    wu = wu_ref[0]
    wd = wd_ref[0]

    def chunk(i, carry):
        base = pl.multiple_of(start + i * CHUNK, 8)
        rows = xs_ref[pl.ds(base, CHUNK), :]
        g = lax.dot_general(rows, wg, (((1,), (1,)), ((), ())),
                            preferred_element_type=jnp.float32)
        u = lax.dot_general(rows, wu, (((1,), (1,)), ((), ())),
                            preferred_element_type=jnp.float32)
        h = _gelu(g) * u
        part = lax.dot_general(h, wd, (((1,), (1,)), ((), ())),
                               preferred_element_type=jnp.float32)
        part = part * ws_ref[pl.ds(base, CHUNK), :]

        @pl.when(it == 0)
        def _():
            ys_ref[pl.ds(base, CHUNK), :] = part

        @pl.when(it > 0)
        def _():
            ys_ref[pl.ds(base, CHUNK), :] += part

        return carry

    lax.fori_loop(0, nch, chunk, 0)


def _tc_grouped(poff, xs, ws, Wg, Wu, Wd):
    return pl.pallas_call(
        _tc_body,
        grid=(E, NI),
        in_specs=[
            pl.BlockSpec(memory_space=pltpu.SMEM),
            pl.BlockSpec((P, H), lambda e, it: (0, 0)),
            pl.BlockSpec((P, 1), lambda e, it: (0, 0)),
            pl.BlockSpec((1, IB, H), lambda e, it: (e, it, 0)),
            pl.BlockSpec((1, IB, H), lambda e, it: (e, it, 0)),
            pl.BlockSpec((1, H, IB), lambda e, it: (e, 0, it)),
        ],
        out_specs=pl.BlockSpec((P, H), lambda e, it: (0, 0)),
        out_shape=jax.ShapeDtypeStruct((P, H), jnp.float32),
        compiler_params=pltpu.CompilerParams(
            dimension_semantics=("arbitrary", "arbitrary")),
    )(poff, xs, ws, Wg, Wu, Wd)


def kernel(x, selected_experts, routing_weights, Wg, Wu, Wd):
    fe = selected_experts.reshape(-1).astype(jnp.int32)
    fw = routing_weights.reshape(-1).astype(jnp.float32)

    oh = (fe[:, None] == jnp.arange(E, dtype=jnp.int32)[None, :]).astype(jnp.int32)
    csum = jnp.cumsum(oh, axis=0)
    counts = csum[-1]
    rank = jnp.sum(oh * csum, axis=1) - 1
    pcounts = ((counts + 7) // 8) * 8
    poff = jnp.concatenate([jnp.zeros((1,), jnp.int32),
                            jnp.cumsum(pcounts).astype(jnp.int32)])
    pos = jnp.take(poff, fe) + rank

    tokw = jnp.stack([jnp.arange(T, dtype=jnp.float32), fw], axis=1)
    meta = jnp.full((P, 2), jnp.float32(TRASH)).at[pos].set(tokw)
    srcS = meta[:, 0].astype(jnp.int32)
    ws = meta[:, 1:2]
    ws = jnp.where(srcS[:, None] == TRASH, 0.0, ws)
    gsrc = jnp.where(srcS == TRASH, 0, srcS)
    dest = jnp.where(srcS == TRASH, TRASH + jnp.arange(P, dtype=jnp.int32), srcS)

    sc_gather, sc_scatter = _sc_kernels()
    xs = sc_gather(x, gsrc)
    ys = _tc_grouped(poff, xs, ws, Wg, Wu, Wd)
    out_ext = sc_scatter(ys, dest)
    return out_ext[:T]


# speedup vs baseline: 1.1136x; 1.1136x over previous
import functools

import jax
import jax.numpy as jnp
from jax import lax
from jax.experimental import pallas as pl
from jax.experimental.pallas import tpu as pltpu
from jax.experimental.pallas import tpu_sc as plsc

H = 1024
I = 1024
E = 64
T = 2048
IB = 512
NI = I // IB
CHUNK = 64
NC, NS = 2, 16
NW = NC * NS
P = 2560
BPW = P // NW
TRASH = T


def _gelu(v):
    return 0.5 * v * (1.0 + jnp.tanh(jnp.sqrt(2.0 / jnp.pi) * (v + 0.044715 * v ** 3)))


_SC_CACHE = {}


def _sc_kernels():
    if "gather" in _SC_CACHE:
        return _SC_CACHE["gather"], _SC_CACHE["scatter"]
    mesh = plsc.VectorSubcoreMesh(core_axis_name="c", subcore_axis_name="s")
    scratch = [
        pltpu.VMEM((BPW,), jnp.int32),
        pltpu.VMEM((BPW, H), jnp.float32),
        pltpu.SemaphoreType.DMA,
    ]

    @functools.partial(
        pl.kernel, mesh=mesh,
        out_type=jax.ShapeDtypeStruct((P, H), jnp.float32),
        scratch_types=scratch,
    )
    def _sc_gather(table_hbm, idx_hbm, out_hbm, idx_v, rows_v, sem):
        wid = lax.axis_index("s") * NC + lax.axis_index("c")
        base = wid * BPW
        pltpu.sync_copy(idx_hbm.at[pl.ds(base, BPW)], idx_v)
        pltpu.async_copy(table_hbm.at[idx_v], rows_v, sem).wait()
        pltpu.sync_copy(rows_v, out_hbm.at[pl.ds(base, BPW)])

    @functools.partial(
        pl.kernel, mesh=mesh,
        out_type=jax.ShapeDtypeStruct((T + P, H), jnp.float32),
        scratch_types=scratch,
    )
    def _sc_scatter(rows_hbm, idx_hbm, out_hbm, idx_v, rows_v, sem):
        wid = lax.axis_index("s") * NC + lax.axis_index("c")
        base = wid * BPW
        pltpu.sync_copy(idx_hbm.at[pl.ds(base, BPW)], idx_v)
        pltpu.sync_copy(rows_hbm.at[pl.ds(base, BPW)], rows_v)
        pltpu.async_copy(rows_v, out_hbm.at[idx_v], sem).wait()

    _SC_CACHE["gather"] = _sc_gather
    _SC_CACHE["scatter"] = _sc_scatter
    return _sc_gather, _sc_scatter


def _tc_body(poff_ref, xs_ref, ws_ref, wg_ref, wu_ref, wd_ref, ys_ref):
    e = pl.program_id(0)
    start = poff_ref[e]
    size = poff_ref[e + 1] - start
    nch = (size + CHUNK - 1) // CHUNK
    wg = wg_ref[0]
    wu = wu_ref[0]
    wd = wd_ref[0]

    def chunk(i, carry):
        base = pl.multiple_of(start + i * CHUNK, 8)
        rows = xs_ref[pl.ds(base, CHUNK), :]
        g = lax.dot_general(rows, wg, (((1,), (1,)), ((), ())),
                            preferred_element_type=jnp.float32)
        u = lax.dot_general(rows, wu, (((1,), (1,)), ((), ())),
                            preferred_element_type=jnp.float32)
        h = _gelu(g) * u
        part = lax.dot_general(h, wd, (((1,), (1,)), ((), ())),
                               preferred_element_type=jnp.float32)
        ys_ref[pl.ds(base, CHUNK), :] = part * ws_ref[pl.ds(base, CHUNK), :]
        return carry

    lax.fori_loop(0, nch, chunk, 0)


def _tc_grouped(poff, xs, ws, Wg, Wu, Wd):
    return pl.pallas_call(
        _tc_body,
        grid=(E,),
        in_specs=[
            pl.BlockSpec(memory_space=pltpu.SMEM),
            pl.BlockSpec((P, H), lambda e: (0, 0)),
            pl.BlockSpec((P, 1), lambda e: (0, 0)),
            pl.BlockSpec((1, I, H), lambda e: (e, 0, 0)),
            pl.BlockSpec((1, I, H), lambda e: (e, 0, 0)),
            pl.BlockSpec((1, H, I), lambda e: (e, 0, 0)),
        ],
        out_specs=pl.BlockSpec((P, H), lambda e: (0, 0)),
        out_shape=jax.ShapeDtypeStruct((P, H), jnp.float32),
        compiler_params=pltpu.CompilerParams(
            dimension_semantics=("arbitrary",)),
    )(poff, xs, ws, Wg, Wu, Wd)


def kernel(x, selected_experts, routing_weights, Wg, Wu, Wd):
    fe = selected_experts.reshape(-1).astype(jnp.int32)
    fw = routing_weights.reshape(-1).astype(jnp.float32)

    oh = (fe[:, None] == jnp.arange(E, dtype=jnp.int32)[None, :]).astype(jnp.int32)
    csum = jnp.cumsum(oh, axis=0)
    counts = csum[-1]
    rank = jnp.sum(oh * csum, axis=1) - 1
    pcounts = ((counts + 7) // 8) * 8
    poff = jnp.concatenate([jnp.zeros((1,), jnp.int32),
                            jnp.cumsum(pcounts).astype(jnp.int32)])
    pos = jnp.take(poff, fe) + rank

    tokw = jnp.stack([jnp.arange(T, dtype=jnp.float32), fw], axis=1)
    meta = jnp.full((P, 2), jnp.float32(TRASH)).at[pos].set(tokw)
    srcS = meta[:, 0].astype(jnp.int32)
    ws = meta[:, 1:2]
    ws = jnp.where(srcS[:, None] == TRASH, 0.0, ws)
    gsrc = jnp.where(srcS == TRASH, 0, srcS)
    dest = jnp.where(srcS == TRASH, TRASH + jnp.arange(P, dtype=jnp.int32), srcS)

    sc_gather, sc_scatter = _sc_kernels()
    xs = sc_gather(x, gsrc)
    ys = _tc_grouped(poff, xs, ws, Wg, Wu, Wd)
    out_ext = sc_scatter(ys, dest)
    return out_ext[:T]


# speedup vs baseline: 1.2417x; 1.1151x over previous
import functools

import jax
import jax.numpy as jnp
from jax import lax
from jax.experimental import pallas as pl
from jax.experimental.pallas import tpu as pltpu
from jax.experimental.pallas import tpu_sc as plsc

H = 1024
I = 1024
E = 64
T = 2048
CHUNK = 64
NC, NS = 2, 16
NW = NC * NS
TPW = T // NW
P = 2560


def _gelu(v):
    return 0.5 * v * (1.0 + jnp.tanh(jnp.sqrt(2.0 / jnp.pi) * (v + 0.044715 * v ** 3)))


_SC_CACHE = {}


def _sc_kernels():
    if "dispatch" in _SC_CACHE:
        return _SC_CACHE["dispatch"], _SC_CACHE["collect"]
    mesh = plsc.VectorSubcoreMesh(core_axis_name="c", subcore_axis_name="s")

    @functools.partial(
        pl.kernel, mesh=mesh,
        out_type=(jax.ShapeDtypeStruct((P, H), jnp.float32),
                  jax.ShapeDtypeStruct((P,), jnp.float32)),
        scratch_types=[
            pltpu.VMEM((TPW,), jnp.int32),
            pltpu.VMEM((TPW,), jnp.float32),
            pltpu.VMEM((TPW, H), jnp.float32),
            pltpu.SemaphoreType.DMA,
            pltpu.SemaphoreType.DMA,
        ],
    )
    def _sc_dispatch(x_hbm, pos_hbm, fw_hbm, xs_hbm, ws_hbm,
                     pos_v, fw_v, rows_v, sem, semw):
        wid = lax.axis_index("s") * NC + lax.axis_index("c")
        base = wid * TPW
        pltpu.sync_copy(pos_hbm.at[pl.ds(base, TPW)], pos_v)
        pltpu.sync_copy(fw_hbm.at[pl.ds(base, TPW)], fw_v)
        pltpu.sync_copy(x_hbm.at[pl.ds(base, TPW)], rows_v)
        a = pltpu.async_copy(rows_v, xs_hbm.at[pos_v], sem)
        b = pltpu.async_copy(fw_v, ws_hbm.at[pos_v], semw)
        a.wait()
        b.wait()

    @functools.partial(
        pl.kernel, mesh=mesh,
        out_type=jax.ShapeDtypeStruct((T, H), jnp.float32),
        scratch_types=[
            pltpu.VMEM((TPW,), jnp.int32),
            pltpu.VMEM((TPW, H), jnp.float32),
            pltpu.SemaphoreType.DMA,
        ],
    )
    def _sc_collect(ys_hbm, pos_hbm, out_hbm, pos_v, rows_v, sem):
        wid = lax.axis_index("s") * NC + lax.axis_index("c")
        base = wid * TPW
        pltpu.sync_copy(pos_hbm.at[pl.ds(base, TPW)], pos_v)
        pltpu.async_copy(ys_hbm.at[pos_v], rows_v, sem).wait()
        pltpu.sync_copy(rows_v, out_hbm.at[pl.ds(base, TPW)])

    _SC_CACHE["dispatch"] = _sc_dispatch
    _SC_CACHE["collect"] = _sc_collect
    return _sc_dispatch, _sc_collect


def _tc_body(poff_ref, xs_ref, ws_ref, wg_ref, wu_ref, wd_ref, ys_ref):
    e = pl.program_id(0)
    start = poff_ref[e]
    size = poff_ref[e + 1] - start
    nch = (size + CHUNK - 1) // CHUNK
    wg = wg_ref[0]
    wu = wu_ref[0]
    wd = wd_ref[0]

    def chunk(i, carry):
        base = pl.multiple_of(start + i * CHUNK, 8)
        rows = xs_ref[pl.ds(base, CHUNK), :]
        g = lax.dot_general(rows, wg, (((1,), (1,)), ((), ())),
                            preferred_element_type=jnp.float32)
        u = lax.dot_general(rows, wu, (((1,), (1,)), ((), ())),
                            preferred_element_type=jnp.float32)
        h = _gelu(g) * u
        part = lax.dot_general(h, wd, (((1,), (1,)), ((), ())),
                               preferred_element_type=jnp.float32)
        ys_ref[pl.ds(base, CHUNK), :] = part * ws_ref[pl.ds(base, CHUNK), :]
        return carry

    lax.fori_loop(0, nch, chunk, 0)


def _tc_grouped(poff, xs, ws, Wg, Wu, Wd):
    return pl.pallas_call(
        _tc_body,
        grid=(E,),
        in_specs=[
            pl.BlockSpec(memory_space=pltpu.SMEM),
            pl.BlockSpec((P, H), lambda e: (0, 0)),
            pl.BlockSpec((P, 1), lambda e: (0, 0)),
            pl.BlockSpec((1, I, H), lambda e: (e, 0, 0)),
            pl.BlockSpec((1, I, H), lambda e: (e, 0, 0)),
            pl.BlockSpec((1, H, I), lambda e: (e, 0, 0)),
        ],
        out_specs=pl.BlockSpec((P, H), lambda e: (0, 0)),
        out_shape=jax.ShapeDtypeStruct((P, H), jnp.float32),
        compiler_params=pltpu.CompilerParams(
            dimension_semantics=("arbitrary",)),
    )(poff, xs, ws, Wg, Wu, Wd)


def _positions(fe, dtype=jnp.int32):
    NB = 16
    BL = T // NB
    oh3 = (fe.reshape(NB, BL)[:, :, None]
           == jnp.arange(E, dtype=jnp.int32)).astype(jnp.float32)
    tril = jnp.tril(jnp.ones((BL, BL), jnp.float32))
    inner = lax.dot_general(tril, oh3, (((1,), (1,)), ((), ())),
                            preferred_element_type=jnp.float32)
    inner = inner.transpose(1, 0, 2)
    blocks = jnp.sum(oh3, axis=1)
    bpref = jnp.cumsum(blocks, axis=0) - blocks
    csum3 = inner + bpref[:, None, :]
    rank = jnp.sum(oh3 * csum3, axis=-1).reshape(T) - 1.0
    counts = jnp.sum(blocks, axis=0).astype(dtype)
    pcounts = ((counts + 7) // 8) * 8
    poff = jnp.concatenate([jnp.zeros((1,), dtype),
                            jnp.cumsum(pcounts).astype(dtype)])
    pos = jnp.take(poff, fe) + rank.astype(dtype)
    return pos, poff


def kernel(x, selected_experts, routing_weights, Wg, Wu, Wd):
    fe = selected_experts.reshape(-1).astype(jnp.int32)
    fw = routing_weights.reshape(-1).astype(jnp.float32)
    pos, poff = _positions(fe)

    sc_dispatch, sc_collect = _sc_kernels()
    xs, ws = sc_dispatch(x, pos, fw)
    ys = _tc_grouped(poff, xs, ws.reshape(P, 1), Wg, Wu, Wd)
    return sc_collect(ys, pos)


# speedup vs baseline: 1.3256x; 1.0675x over previous
import functools

import jax
import jax.numpy as jnp
from jax import lax
from jax.experimental import pallas as pl
from jax.experimental.pallas import tpu as pltpu
from jax.experimental.pallas import tpu_sc as plsc

H = 1024
I = 1024
E = 64
T = 2048
CHUNK = 64
NC, NS = 2, 16
NW = NC * NS
TPW = T // NW
P = 2560


def _gelu(v):
    return 0.5 * v * (1.0 + jnp.tanh(jnp.sqrt(2.0 / jnp.pi) * (v + 0.044715 * v ** 3)))


_SC_CACHE = {}


def _sc_kernels():
    if "dispatch" in _SC_CACHE:
        return _SC_CACHE["dispatch"], _SC_CACHE["collect"]
    mesh = plsc.VectorSubcoreMesh(core_axis_name="c", subcore_axis_name="s")

    @functools.partial(
        pl.kernel, mesh=mesh,
        out_type=jax.ShapeDtypeStruct((P, H), jnp.float32),
        scratch_types=[
            pltpu.VMEM((TPW,), jnp.int32),
            pltpu.VMEM((TPW, H), jnp.float32),
            pltpu.SemaphoreType.DMA,
        ],
    )
    def _sc_dispatch(x_hbm, pos_hbm, xs_hbm, pos_v, rows_v, sem):
        wid = lax.axis_index("s") * NC + lax.axis_index("c")
        base = wid * TPW
        pltpu.sync_copy(pos_hbm.at[pl.ds(base, TPW)], pos_v)
        pltpu.sync_copy(x_hbm.at[pl.ds(base, TPW)], rows_v)
        pltpu.async_copy(rows_v, xs_hbm.at[pos_v], sem).wait()

    @functools.partial(
        pl.kernel, mesh=mesh,
        out_type=jax.ShapeDtypeStruct((T, H), jnp.float32),
        scratch_types=[
            pltpu.VMEM((TPW,), jnp.int32),
            pltpu.VMEM((TPW, H), jnp.float32),
            pltpu.SemaphoreType.DMA,
        ],
    )
    def _sc_collect(ys_hbm, pos_hbm, out_hbm, pos_v, rows_v, sem):
        wid = lax.axis_index("s") * NC + lax.axis_index("c")
        base = wid * TPW
        pltpu.sync_copy(pos_hbm.at[pl.ds(base, TPW)], pos_v)
        pltpu.async_copy(ys_hbm.at[pos_v], rows_v, sem).wait()
        pltpu.sync_copy(rows_v, out_hbm.at[pl.ds(base, TPW)])

    _SC_CACHE["dispatch"] = _sc_dispatch
    _SC_CACHE["collect"] = _sc_collect
    return _sc_dispatch, _sc_collect


def _tc_body(poff_ref, xs_ref, ws_ref, wg_ref, wu_ref, wd_ref, ys_ref):
    e = pl.program_id(0)
    start = poff_ref[e]
    size = poff_ref[e + 1] - start
    nch = (size + CHUNK - 1) // CHUNK
    wg = wg_ref[0]
    wu = wu_ref[0]
    wd = wd_ref[0]

    def chunk(i, carry):
        base = pl.multiple_of(start + i * CHUNK, 8)
        rows = xs_ref[pl.ds(base, CHUNK), :]
        g = lax.dot_general(rows, wg, (((1,), (1,)), ((), ())),
                            preferred_element_type=jnp.float32)
        u = lax.dot_general(rows, wu, (((1,), (1,)), ((), ())),
                            preferred_element_type=jnp.float32)
        h = _gelu(g) * u
        part = lax.dot_general(h, wd, (((1,), (1,)), ((), ())),
                               preferred_element_type=jnp.float32)
        ys_ref[pl.ds(base, CHUNK), :] = part * ws_ref[pl.ds(base, CHUNK), :]
        return carry

    lax.fori_loop(0, nch, chunk, 0)


def _tc_grouped(poff, xs, ws, Wg, Wu, Wd):
    return pl.pallas_call(
        _tc_body,
        grid=(E,),
        in_specs=[
            pl.BlockSpec(memory_space=pltpu.SMEM),
            pl.BlockSpec((P, H), lambda e: (0, 0)),
            pl.BlockSpec((P, 1), lambda e: (0, 0)),
            pl.BlockSpec((1, I, H), lambda e: (e, 0, 0)),
            pl.BlockSpec((1, I, H), lambda e: (e, 0, 0)),
            pl.BlockSpec((1, H, I), lambda e: (e, 0, 0)),
        ],
        out_specs=pl.BlockSpec((P, H), lambda e: (0, 0)),
        out_shape=jax.ShapeDtypeStruct((P, H), jnp.float32),
        compiler_params=pltpu.CompilerParams(
            dimension_semantics=("arbitrary",)),
    )(poff, xs, ws, Wg, Wu, Wd)


def _positions(fe, dtype=jnp.int32):
    NB = 16
    BL = T // NB
    oh3 = (fe.reshape(NB, BL)[:, :, None]
           == jnp.arange(E, dtype=jnp.int32)).astype(jnp.float32)
    tril = jnp.tril(jnp.ones((BL, BL), jnp.float32))
    inner = lax.dot_general(tril, oh3, (((1,), (1,)), ((), ())),
                            preferred_element_type=jnp.float32)
    inner = inner.transpose(1, 0, 2)
    blocks = jnp.sum(oh3, axis=1)
    bpref = jnp.cumsum(blocks, axis=0) - blocks
    csum3 = inner + bpref[:, None, :]
    rank = jnp.sum(oh3 * csum3, axis=-1).reshape(T) - 1.0
    counts = jnp.sum(blocks, axis=0).astype(dtype)
    pcounts = ((counts + 7) // 8) * 8
    poff = jnp.concatenate([jnp.zeros((1,), dtype),
                            jnp.cumsum(pcounts).astype(dtype)])
    pos = jnp.take(poff, fe) + rank.astype(dtype)
    return pos, poff


def kernel(x, selected_experts, routing_weights, Wg, Wu, Wd):
    fe = selected_experts.reshape(-1).astype(jnp.int32)
    fw = routing_weights.reshape(-1).astype(jnp.float32)
    pos, poff = _positions(fe)

    sc_dispatch, sc_collect = _sc_kernels()
    xs = sc_dispatch(x, pos)
    ws = jnp.zeros((P, 1), jnp.float32).at[pos, 0].set(fw)
    ys = _tc_grouped(poff, xs, ws, Wg, Wu, Wd)
    return sc_collect(ys, pos)


# speedup vs baseline: 1.4229x; 1.0734x over previous
import functools

import jax
import jax.numpy as jnp
from jax import lax
from jax.experimental import pallas as pl
from jax.experimental.pallas import tpu as pltpu
from jax.experimental.pallas import tpu_sc as plsc

H = 1024
I = 1024
E = 64
T = 2048
CHUNK = 64
NC, NS = 2, 16
NW = NC * NS
TPW = T // NW
P = 2560


def _gelu(v):
    return 0.5 * v * (1.0 + jnp.tanh(jnp.sqrt(2.0 / jnp.pi) * (v + 0.044715 * v ** 3)))


_SC_CACHE = {}


def _sc_kernels():
    if "dispatch" in _SC_CACHE:
        return _SC_CACHE["dispatch"], _SC_CACHE["collect"]
    mesh = plsc.VectorSubcoreMesh(core_axis_name="c", subcore_axis_name="s")

    @functools.partial(
        pl.kernel, mesh=mesh,
        out_type=jax.ShapeDtypeStruct((P, H), jnp.float32),
        scratch_types=[
            pltpu.VMEM((TPW,), jnp.int32),
            pltpu.VMEM((TPW, H), jnp.float32),
            pltpu.SemaphoreType.DMA,
        ],
    )
    def _sc_dispatch(x_hbm, pos_hbm, xs_hbm, pos_v, rows_v, sem):
        wid = lax.axis_index("s") * NC + lax.axis_index("c")
        base = wid * TPW
        pltpu.sync_copy(pos_hbm.at[pl.ds(base, TPW)], pos_v)
        pltpu.sync_copy(x_hbm.at[pl.ds(base, TPW)], rows_v)
        pltpu.async_copy(rows_v, xs_hbm.at[pos_v], sem).wait()

    @functools.partial(
        pl.kernel, mesh=mesh,
        out_type=jax.ShapeDtypeStruct((T, H), jnp.float32),
        scratch_types=[
            pltpu.VMEM((TPW,), jnp.int32),
            pltpu.VMEM((TPW, H), jnp.float32),
            pltpu.SemaphoreType.DMA,
        ],
    )
    def _sc_collect(ys_hbm, pos_hbm, out_hbm, pos_v, rows_v, sem):
        wid = lax.axis_index("s") * NC + lax.axis_index("c")
        base = wid * TPW
        pltpu.sync_copy(pos_hbm.at[pl.ds(base, TPW)], pos_v)
        pltpu.async_copy(ys_hbm.at[pos_v], rows_v, sem).wait()
        pltpu.sync_copy(rows_v, out_hbm.at[pl.ds(base, TPW)])

    _SC_CACHE["dispatch"] = _sc_dispatch
    _SC_CACHE["collect"] = _sc_collect
    return _sc_dispatch, _sc_collect


def _tc_body(poff_ref, xs_ref, ws_ref, wga_ref, wgb_ref, wua_ref, wub_ref,
             wda_ref, wdb_ref, ys_ref):
    e = pl.program_id(0)
    start = poff_ref[e]
    size = poff_ref[e + 1] - start
    nch = (size + CHUNK - 1) // CHUNK

    def dots(rows, a_ref, b_ref):
        pa = lax.dot_general(rows, a_ref[0], (((1,), (1,)), ((), ())),
                             preferred_element_type=jnp.float32)
        pb = lax.dot_general(rows, b_ref[0], (((1,), (1,)), ((), ())),
                             preferred_element_type=jnp.float32)
        return jnp.concatenate([pa, pb], axis=1)

    def chunk(i, carry):
        base = pl.multiple_of(start + i * CHUNK, 8)
        rows = xs_ref[pl.ds(base, CHUNK), :]
        g = dots(rows, wga_ref, wgb_ref)
        u = dots(rows, wua_ref, wub_ref)
        h = _gelu(g) * u
        part = dots(h, wda_ref, wdb_ref)
        ys_ref[pl.ds(base, CHUNK), :] = part * ws_ref[pl.ds(base, CHUNK), :]
        return carry

    lax.fori_loop(0, nch, chunk, 0)


def _tc_grouped(poff, xs, ws, Wg, Wu, Wd):
    half = pl.BlockSpec((1, I // 2, H), lambda e: (e, 0, 0))
    halfb = pl.BlockSpec((1, I // 2, H), lambda e: (e, 1, 0))
    return pl.pallas_call(
        _tc_body,
        grid=(E,),
        in_specs=[
            pl.BlockSpec(memory_space=pltpu.SMEM),
            pl.BlockSpec((P, H), lambda e: (0, 0)),
            pl.BlockSpec((P, 1), lambda e: (0, 0)),
            half, halfb, half, halfb, half, halfb,
        ],
        out_specs=pl.BlockSpec((P, H), lambda e: (0, 0)),
        out_shape=jax.ShapeDtypeStruct((P, H), jnp.float32),
        compiler_params=pltpu.CompilerParams(
            dimension_semantics=("arbitrary",)),
    )(poff, xs, ws, Wg, Wg, Wu, Wu, Wd, Wd)


def _positions(fe, dtype=jnp.int32):
    NB = 16
    BL = T // NB
    oh3 = (fe.reshape(NB, BL)[:, :, None]
           == jnp.arange(E, dtype=jnp.int32)).astype(jnp.float32)
    tril = jnp.tril(jnp.ones((BL, BL), jnp.float32))
    inner = lax.dot_general(tril, oh3, (((1,), (1,)), ((), ())),
                            preferred_element_type=jnp.float32)
    inner = inner.transpose(1, 0, 2)
    blocks = jnp.sum(oh3, axis=1)
    bpref = jnp.cumsum(blocks, axis=0) - blocks
    csum3 = inner + bpref[:, None, :]
    rank = jnp.sum(oh3 * csum3, axis=-1).reshape(T) - 1.0
    counts = jnp.sum(blocks, axis=0).astype(dtype)
    pcounts = ((counts + 7) // 8) * 8
    poff = jnp.concatenate([jnp.zeros((1,), dtype),
                            jnp.cumsum(pcounts).astype(dtype)])
    pos = jnp.take(poff, fe) + rank.astype(dtype)
    return pos, poff


def kernel(x, selected_experts, routing_weights, Wg, Wu, Wd):
    fe = selected_experts.reshape(-1).astype(jnp.int32)
    fw = routing_weights.reshape(-1).astype(jnp.float32)
    pos, poff = _positions(fe)

    sc_dispatch, sc_collect = _sc_kernels()
    xs = sc_dispatch(x, pos)
    ws = jnp.zeros((P, 1), jnp.float32).at[pos, 0].set(fw)
    ys = _tc_grouped(poff, xs, ws, Wg, Wu, Wd)
    return sc_collect(ys, pos)


# speedup vs baseline: 1.4369x; 1.0098x over previous
import functools

import jax
import jax.numpy as jnp
from jax import lax
from jax.experimental import pallas as pl
from jax.experimental.pallas import tpu as pltpu
from jax.experimental.pallas import tpu_sc as plsc

H = 1024
I = 1024
E = 64
T = 2048
CHUNK = 64
NC, NS = 2, 16
NW = NC * NS
TPW = T // NW
P = 2560


def _gelu(v):
    return 0.5 * v * (1.0 + jnp.tanh(jnp.sqrt(2.0 / jnp.pi) * (v + 0.044715 * v ** 3)))


_SC_CACHE = {}


def _sc_kernels():
    if "dispatch" in _SC_CACHE:
        return _SC_CACHE["dispatch"], _SC_CACHE["collect"]
    mesh = plsc.VectorSubcoreMesh(core_axis_name="c", subcore_axis_name="s")

    @functools.partial(
        pl.kernel, mesh=mesh,
        out_type=jax.ShapeDtypeStruct((P, H), jnp.float32),
        scratch_types=[
            pltpu.VMEM((TPW,), jnp.int32),
            pltpu.VMEM((TPW, H), jnp.float32),
            pltpu.SemaphoreType.DMA,
        ],
    )
    def _sc_dispatch(x_hbm, pos_hbm, xs_hbm, pos_v, rows_v, sem):
        wid = lax.axis_index("s") * NC + lax.axis_index("c")
        base = wid * TPW
        pltpu.sync_copy(pos_hbm.at[pl.ds(base, TPW)], pos_v)
        pltpu.sync_copy(x_hbm.at[pl.ds(base, TPW)], rows_v)
        pltpu.async_copy(rows_v, xs_hbm.at[pos_v], sem).wait()

    @functools.partial(
        pl.kernel, mesh=mesh,
        out_type=jax.ShapeDtypeStruct((T, H), jnp.float32),
        scratch_types=[
            pltpu.VMEM((TPW,), jnp.int32),
            pltpu.VMEM((TPW, H), jnp.float32),
            pltpu.SemaphoreType.DMA,
        ],
    )
    def _sc_collect(ys_hbm, pos_hbm, out_hbm, pos_v, rows_v, sem):
        wid = lax.axis_index("s") * NC + lax.axis_index("c")
        base = wid * TPW
        pltpu.sync_copy(pos_hbm.at[pl.ds(base, TPW)], pos_v)
        pltpu.async_copy(ys_hbm.at[pos_v], rows_v, sem).wait()
        pltpu.sync_copy(rows_v, out_hbm.at[pl.ds(base, TPW)])

    _SC_CACHE["dispatch"] = _sc_dispatch
    _SC_CACHE["collect"] = _sc_collect
    return _sc_dispatch, _sc_collect


NSPL = 4


def _tc_body(poff_ref, xs_ref, ws_ref, *refs):
    w_refs, ys_ref = refs[:3 * NSPL], refs[3 * NSPL]
    e = pl.program_id(0)
    start = poff_ref[e]
    size = poff_ref[e + 1] - start
    nch = (size + CHUNK - 1) // CHUNK

    def dots(rows, parts):
        outs = [lax.dot_general(rows, r[0], (((1,), (1,)), ((), ())),
                                preferred_element_type=jnp.float32)
                for r in parts]
        return jnp.concatenate(outs, axis=1)

    def chunk(i, carry):
        base = pl.multiple_of(start + i * CHUNK, 8)
        rows = xs_ref[pl.ds(base, CHUNK), :]
        g = dots(rows, w_refs[:NSPL])
        u = dots(rows, w_refs[NSPL:2 * NSPL])
        h = _gelu(g) * u
        part = dots(h, w_refs[2 * NSPL:])
        ys_ref[pl.ds(base, CHUNK), :] = part * ws_ref[pl.ds(base, CHUNK), :]
        return carry

    lax.fori_loop(0, nch, chunk, 0)


def _tc_grouped(poff, xs, ws, Wg, Wu, Wd):
    w_specs = [pl.BlockSpec((1, I // NSPL, H), lambda e, j=j: (e, j, 0))
               for j in range(NSPL)]
    return pl.pallas_call(
        _tc_body,
        grid=(E,),
        in_specs=[
            pl.BlockSpec(memory_space=pltpu.SMEM),
            pl.BlockSpec((P, H), lambda e: (0, 0)),
            pl.BlockSpec((P, 1), lambda e: (0, 0)),
            *w_specs, *w_specs, *w_specs,
        ],
        out_specs=pl.BlockSpec((P, H), lambda e: (0, 0)),
        out_shape=jax.ShapeDtypeStruct((P, H), jnp.float32),
        compiler_params=pltpu.CompilerParams(
            dimension_semantics=("arbitrary",)),
    )(poff, xs, ws, *([Wg] * NSPL), *([Wu] * NSPL), *([Wd] * NSPL))


def _positions(fe, dtype=jnp.int32):
    NB = 16
    BL = T // NB
    oh3 = (fe.reshape(NB, BL)[:, :, None]
           == jnp.arange(E, dtype=jnp.int32)).astype(jnp.float32)
    tril = jnp.tril(jnp.ones((BL, BL), jnp.float32))
    inner = lax.dot_general(tril, oh3, (((1,), (1,)), ((), ())),
                            preferred_element_type=jnp.float32)
    inner = inner.transpose(1, 0, 2)
    blocks = jnp.sum(oh3, axis=1)
    bpref = jnp.cumsum(blocks, axis=0) - blocks
    csum3 = inner + bpref[:, None, :]
    rank = jnp.sum(oh3 * csum3, axis=-1).reshape(T) - 1.0
    counts = jnp.sum(blocks, axis=0).astype(dtype)
    pcounts = ((counts + 7) // 8) * 8
    poff = jnp.concatenate([jnp.zeros((1,), dtype),
                            jnp.cumsum(pcounts).astype(dtype)])
    pos = jnp.take(poff, fe) + rank.astype(dtype)
    return pos, poff


def kernel(x, selected_experts, routing_weights, Wg, Wu, Wd):
    fe = selected_experts.reshape(-1).astype(jnp.int32)
    fw = routing_weights.reshape(-1).astype(jnp.float32)
    pos, poff = _positions(fe)

    sc_dispatch, sc_collect = _sc_kernels()
    xs = sc_dispatch(x, pos)
    ws = jnp.zeros((P, 1), jnp.float32).at[pos, 0].set(fw)
    ys = _tc_grouped(poff, xs, ws, Wg, Wu, Wd)
    return sc_collect(ys, pos)
